# baseline (speedup 1.0000x reference)
import functools

import jax
import jax.numpy as jnp
from jax import lax
from jax.experimental import pallas as pl
from jax.experimental.pallas import tpu as pltpu
from jax.experimental.pallas import tpu_sc as plsc

N = 1024
E = 32768
DH = 128
NW = 32
RPW = N // NW
LANES = 16
BLK = 128
NB = N // BLK

_HIGH = lax.Precision.HIGHEST


def _build_adjacency(edge_index):
    mesh = plsc.VectorSubcoreMesh(core_axis_name="c", subcore_axis_name="s")

    @functools.partial(
        pl.kernel,
        out_type=jax.ShapeDtypeStruct((NW, RPW * N), jnp.float32),
        mesh=mesh,
        compiler_params=pltpu.CompilerParams(needs_layout_passes=False),
        scratch_types=[
            pltpu.VMEM((E,), jnp.int32),
            pltpu.VMEM((E,), jnp.int32),
            pltpu.VMEM((RPW * N,), jnp.float32),
        ],
    )
    def adj(edges_hbm, out_hbm, row_v, col_v, slab):
        wid = lax.axis_index("s") * 2 + lax.axis_index("c")
        lo = wid * RPW
        pltpu.sync_copy(edges_hbm.at[0], row_v)
        pltpu.sync_copy(edges_hbm.at[1], col_v)
        zeros16 = jnp.zeros((LANES,), jnp.float32)
        ones16 = jnp.ones((LANES,), jnp.float32)
        def zbody(k, _):
            slab[pl.ds(k * LANES, LANES)] = zeros16
            return 0
        lax.fori_loop(0, RPW * N // LANES, zbody, 0)
        j16 = lax.iota(jnp.int32, LANES)
        for h in range(RPW // LANES):
            jj = j16 + h * LANES
            plsc.store_scatter(slab, [jj * (N + 1) + lo], ones16)
        def ebody(i, _):
            r16 = row_v[pl.ds(i * LANES, LANES)]
            c16 = col_v[pl.ds(i * LANES, LANES)]
            msk = (c16 >= lo) & (c16 < lo + RPW)
            plsc.addupdate_scatter(slab, [(c16 - lo) * N + r16], ones16, mask=msk)
            return 0
        lax.fori_loop(0, E // LANES, ebody, 0)
        pltpu.sync_copy(slab, out_hbm.at[wid])

    return adj(edge_index).reshape(N, N)


def _dense_chain(A, x, Wc0, bc0, Wc1, bc1, Wc2, bc2, Wout, bout, Wp1, bp1):
    def body(a_ref, x_ref, w0, b0, w1, b1, w2, b2, wo, bo, wp1, bp1_ref,
             emb_ref, pa_ref, pbt_ref):
        A = a_ref[...]
        deg = jnp.sum(A, axis=1, keepdims=True)
        dinv = lax.rsqrt(deg)
        h = x_ref[...]
        for (w, b, act) in ((w0, b0, True), (w1, b1, True), (w2, b2, False)):
            t = jnp.dot(h, w[...], precision=_HIGH, preferred_element_type=jnp.float32)
            t = t * dinv
            t = jnp.dot(A, t, precision=_HIGH, preferred_element_type=jnp.float32)
            t = t * dinv + b[...]
            h = jnp.maximum(t, 0.0) if act else t
        emb = jnp.dot(h, wo[...], precision=_HIGH,
                      preferred_element_type=jnp.float32) + bo[...]
        emb_ref[...] = emb
        wp = wp1[...]
        pa_ref[...] = jnp.dot(emb, wp[:DH], precision=_HIGH,
                              preferred_element_type=jnp.float32) + bp1_ref[...]
        pb = jnp.dot(emb, wp[DH:], precision=_HIGH,
                     preferred_element_type=jnp.float32)
        pbt_ref[...] = pb.T

    return pl.pallas_call(
        body,
        out_shape=(
            jax.ShapeDtypeStruct((N, DH), jnp.float32),
            jax.ShapeDtypeStruct((N, DH), jnp.float32),
            jax.ShapeDtypeStruct((DH, N), jnp.float32),
        ),
    )(A, x, Wc0, bc0, Wc1, bc1, Wc2, bc2, Wout, bout, Wp1, bp1)


def _pairwise_dist(PA, PBT, Wp2, bp2):
    def body(pa_ref, pbt2_ref, w_ref, b_ref, out_ref, acc_ref):
        bi = pl.program_id(0)

        def rowblk_body(rb, _):
            s = [jnp.zeros((8, BLK), jnp.float32) for _ in range(8)]
            for kc in range(DH // 16):
                bvs = [pbt2_ref[pl.ds((kc * 16 + t) * 8, 8), :]
                       for t in range(16)]
                for row in range(8):
                    for t in range(16):
                        k = kc * 16 + t
                        m = jnp.maximum(bvs[t] + pa_ref[rb * 8 + row, k], 0.0)
                        s[row] = s[row] + m * w_ref[k, 0]
            for row in range(8):
                acc_ref[rb * 8 + row] = s[row]
            return 0

        lax.fori_loop(0, BLK // 8, rowblk_body, 0)
        rr = lax.broadcasted_iota(jnp.int32, (BLK, BLK), 0)
        cc = lax.broadcasted_iota(jnp.int32, (BLK, BLK), 1)
        for bj in range(NB):

            @pl.when(bj >= bi)
            def _(bj=bj):
                blk = jax.nn.sigmoid(
                    acc_ref[:, bj].reshape(BLK, BLK) + b_ref[0, 0])
                blk = jnp.where((bi != bj) | (cc > rr), blk, 0.0)

                @pl.when(bi == bj)
                def _():
                    out_ref[pl.ds(bi * BLK, BLK), pl.ds(bi * BLK, BLK)] = (
                        blk + blk.T)

                @pl.when(bi != bj)
                def _():
                    out_ref[pl.ds(bi * BLK, BLK), pl.ds(bj * BLK, BLK)] = blk
                    out_ref[pl.ds(bj * BLK, BLK), pl.ds(bi * BLK, BLK)] = blk.T

    return pl.pallas_call(
        body,
        grid=(NB,),
        in_specs=[
            pl.BlockSpec((BLK, DH), lambda i: (i, 0), memory_space=pltpu.SMEM),
            pl.BlockSpec((DH * 8, BLK), lambda i: (0, 0)),
            pl.BlockSpec(memory_space=pltpu.SMEM),
            pl.BlockSpec(memory_space=pltpu.SMEM),
        ],
        out_specs=pl.BlockSpec((N, N), lambda i: (0, 0)),
        out_shape=jax.ShapeDtypeStruct((N, N), jnp.float32),
        scratch_shapes=[pltpu.VMEM((BLK, NB, BLK), jnp.float32)],
    )(PA, PBT.reshape(DH * 8, BLK), Wp2, bp2)


def kernel(x, edge_index, Wc0, bc0, Wc1, bc1, Wc2, bc2, Wout, bout, Wp1, bp1, Wp2, bp2):
    A = _build_adjacency(edge_index)
    emb, PA, PBT = _dense_chain(
        A, x, Wc0, bc0.reshape(1, DH), Wc1, bc1.reshape(1, DH),
        Wc2, bc2.reshape(1, DH), Wout, bout.reshape(1, DH),
        Wp1, bp1.reshape(1, DH))
    dist = _pairwise_dist(PA, PBT, Wp2, bp2.reshape(1, 1))
    return emb, dist


# speedup vs baseline: 1.0012x; 1.0012x over previous
import functools

import jax
import jax.numpy as jnp
from jax import lax
from jax.experimental import pallas as pl
from jax.experimental.pallas import tpu as pltpu
from jax.experimental.pallas import tpu_sc as plsc

N = 1024
E = 32768
DH = 128
NW = 32
RPW = N // NW
LANES = 16
BLK = 128
NB = N // BLK

_HIGH = lax.Precision.HIGHEST


def _build_adjacency(edge_index):
    mesh = plsc.VectorSubcoreMesh(core_axis_name="c", subcore_axis_name="s")

    @functools.partial(
        pl.kernel,
        out_type=jax.ShapeDtypeStruct((NW, RPW * N), jnp.float32),
        mesh=mesh,
        compiler_params=pltpu.CompilerParams(needs_layout_passes=False),
        scratch_types=[
            pltpu.VMEM((E,), jnp.int32),
            pltpu.VMEM((E,), jnp.int32),
            pltpu.VMEM((RPW * N,), jnp.float32),
        ],
    )
    def adj(edges_hbm, out_hbm, row_v, col_v, slab):
        wid = lax.axis_index("s") * 2 + lax.axis_index("c")
        lo = wid * RPW
        pltpu.sync_copy(edges_hbm.at[0], row_v)
        pltpu.sync_copy(edges_hbm.at[1], col_v)
        zeros16 = jnp.zeros((LANES,), jnp.float32)
        ones16 = jnp.ones((LANES,), jnp.float32)
        def zbody(k, _):
            slab[pl.ds(k * LANES, LANES)] = zeros16
            return 0
        lax.fori_loop(0, RPW * N // LANES, zbody, 0)
        j16 = lax.iota(jnp.int32, LANES)
        for h in range(RPW // LANES):
            jj = j16 + h * LANES
            plsc.store_scatter(slab, [jj * (N + 1) + lo], ones16)
        def ebody(i, _):
            r16 = row_v[pl.ds(i * LANES, LANES)]
            c16 = col_v[pl.ds(i * LANES, LANES)]
            msk = (c16 >= lo) & (c16 < lo + RPW)
            plsc.addupdate_scatter(slab, [(c16 - lo) * N + r16], ones16, mask=msk)
            return 0
        lax.fori_loop(0, E // LANES, ebody, 0)
        pltpu.sync_copy(slab, out_hbm.at[wid])

    return adj(edge_index).reshape(N, N)


def _dense_chain(A, x, Wc0, bc0, Wc1, bc1, Wc2, bc2, Wout, bout, Wp1, bp1):
    def body(a_ref, x_ref, w0, b0, w1, b1, w2, b2, wo, bo, wp1, bp1_ref,
             emb_ref, pa_ref, pbt_ref):
        A = a_ref[...]
        deg = jnp.sum(A, axis=1, keepdims=True)
        dinv = lax.rsqrt(deg)
        h = x_ref[...]
        for (w, b, act) in ((w0, b0, True), (w1, b1, True), (w2, b2, False)):
            t = jnp.dot(h, w[...], precision=_HIGH, preferred_element_type=jnp.float32)
            t = t * dinv
            t = jnp.dot(A, t, precision=_HIGH, preferred_element_type=jnp.float32)
            t = t * dinv + b[...]
            h = jnp.maximum(t, 0.0) if act else t
        emb = jnp.dot(h, wo[...], precision=_HIGH,
                      preferred_element_type=jnp.float32) + bo[...]
        emb_ref[...] = emb
        wp = wp1[...]
        pa_ref[...] = jnp.dot(emb, wp[:DH], precision=_HIGH,
                              preferred_element_type=jnp.float32) + bp1_ref[...]
        pb = jnp.dot(emb, wp[DH:], precision=_HIGH,
                     preferred_element_type=jnp.float32)
        pbt_ref[...] = pb.T

    return pl.pallas_call(
        body,
        out_shape=(
            jax.ShapeDtypeStruct((N, DH), jnp.float32),
            jax.ShapeDtypeStruct((N, DH), jnp.float32),
            jax.ShapeDtypeStruct((DH, N), jnp.float32),
        ),
    )(A, x, Wc0, bc0, Wc1, bc1, Wc2, bc2, Wout, bout, Wp1, bp1)


def _pairwise_dist(PA, PBT, Wp2, bp2):
    def body(pa_ref, pbt2_ref, w_ref, b_ref, out_ref, acc_ref):
        bi = pl.program_id(0)

        def rowblk_body(rb, _):
            s = [jnp.zeros((8, BLK), jnp.float32) for _ in range(8)]
            for kc in range(DH // 2):
                bvs = [pbt2_ref[pl.ds((kc * 2 + t) * 8, 8), :]
                       for t in range(2)]
                for row in range(8):
                    for t in range(2):
                        k = kc * 2 + t
                        m = jnp.maximum(bvs[t] + pa_ref[rb * 8 + row, k], 0.0)
                        s[row] = s[row] + m * w_ref[k, 0]
            for row in range(8):
                acc_ref[rb * 8 + row] = s[row]
            return 0

        lax.fori_loop(0, BLK // 8, rowblk_body, 0)
        rr = lax.broadcasted_iota(jnp.int32, (BLK, BLK), 0)
        cc = lax.broadcasted_iota(jnp.int32, (BLK, BLK), 1)
        for bj in range(NB):

            @pl.when(bj >= bi)
            def _(bj=bj):
                blk = jax.nn.sigmoid(
                    acc_ref[:, bj].reshape(BLK, BLK) + b_ref[0, 0])
                blk = jnp.where((bi != bj) | (cc > rr), blk, 0.0)

                @pl.when(bi == bj)
                def _():
                    out_ref[pl.ds(bi * BLK, BLK), pl.ds(bi * BLK, BLK)] = (
                        blk + blk.T)

                @pl.when(bi != bj)
                def _():
                    out_ref[pl.ds(bi * BLK, BLK), pl.ds(bj * BLK, BLK)] = blk
                    out_ref[pl.ds(bj * BLK, BLK), pl.ds(bi * BLK, BLK)] = blk.T

    return pl.pallas_call(
        body,
        grid=(NB,),
        in_specs=[
            pl.BlockSpec((BLK, DH), lambda i: (i, 0), memory_space=pltpu.SMEM),
            pl.BlockSpec((DH * 8, BLK), lambda i: (0, 0)),
            pl.BlockSpec(memory_space=pltpu.SMEM),
            pl.BlockSpec(memory_space=pltpu.SMEM),
        ],
        out_specs=pl.BlockSpec((N, N), lambda i: (0, 0)),
        out_shape=jax.ShapeDtypeStruct((N, N), jnp.float32),
        scratch_shapes=[pltpu.VMEM((BLK, NB, BLK), jnp.float32)],
    )(PA, PBT.reshape(DH * 8, BLK), Wp2, bp2)


def kernel(x, edge_index, Wc0, bc0, Wc1, bc1, Wc2, bc2, Wout, bout, Wp1, bp1, Wp2, bp2):
    A = _build_adjacency(edge_index)
    emb, PA, PBT = _dense_chain(
        A, x, Wc0, bc0.reshape(1, DH), Wc1, bc1.reshape(1, DH),
        Wc2, bc2.reshape(1, DH), Wout, bout.reshape(1, DH),
        Wp1, bp1.reshape(1, DH))
    dist = _pairwise_dist(PA, PBT, Wp2, bp2.reshape(1, 1))
    return emb, dist


# speedup vs baseline: 1.2331x; 1.2316x over previous
import functools

import jax
import jax.numpy as jnp
from jax import lax
from jax.experimental import pallas as pl
from jax.experimental.pallas import tpu as pltpu
from jax.experimental.pallas import tpu_sc as plsc

N = 1024
E = 32768
DH = 128
NW = 32
RPW = N // NW
LANES = 16
BLK = 128
NB = N // BLK

def _build_adjacency(edge_index):
    mesh = plsc.VectorSubcoreMesh(core_axis_name="c", subcore_axis_name="s")

    @functools.partial(
        pl.kernel,
        out_type=jax.ShapeDtypeStruct((NW, RPW * N), jnp.float32),
        mesh=mesh,
        compiler_params=pltpu.CompilerParams(needs_layout_passes=False),
        scratch_types=[
            pltpu.VMEM((E,), jnp.int32),
            pltpu.VMEM((E,), jnp.int32),
            pltpu.VMEM((RPW * N,), jnp.float32),
        ],
    )
    def adj(edges_hbm, out_hbm, row_v, col_v, slab):
        wid = lax.axis_index("s") * 2 + lax.axis_index("c")
        lo = wid * RPW
        pltpu.sync_copy(edges_hbm.at[0], row_v)
        pltpu.sync_copy(edges_hbm.at[1], col_v)
        zeros16 = jnp.zeros((LANES,), jnp.float32)
        ones16 = jnp.ones((LANES,), jnp.float32)
        def zbody(k, _):
            slab[pl.ds(k * LANES, LANES)] = zeros16
            return 0
        lax.fori_loop(0, RPW * N // LANES, zbody, 0)
        j16 = lax.iota(jnp.int32, LANES)
        for h in range(RPW // LANES):
            jj = j16 + h * LANES
            plsc.store_scatter(slab, [jj * (N + 1) + lo], ones16)
        def ebody(i, _):
            r16 = row_v[pl.ds(i * LANES, LANES)]
            c16 = col_v[pl.ds(i * LANES, LANES)]
            msk = (c16 >= lo) & (c16 < lo + RPW)
            plsc.addupdate_scatter(slab, [(c16 - lo) * N + r16], ones16, mask=msk)
            return 0
        lax.fori_loop(0, E // LANES, ebody, 0)
        pltpu.sync_copy(slab, out_hbm.at[wid])

    return adj(edge_index).reshape(N, N)


def _dense_chain(A, x, Wc0, bc0, Wc1, bc1, Wc2, bc2, Wout, bout, Wp1, bp1):
    def body(a_ref, x_ref, w0, b0, w1, b1, w2, b2, wo, bo, wp1, bp1_ref,
             emb_ref, pa_ref, pbt_ref):
        A = a_ref[...]
        deg = jnp.sum(A, axis=1, keepdims=True)
        dinv = lax.rsqrt(deg)
        h = x_ref[...]
        for (w, b, act) in ((w0, b0, True), (w1, b1, True), (w2, b2, False)):
            t = jnp.dot(h, w[...], preferred_element_type=jnp.float32)
            t = t * dinv
            t = jnp.dot(A, t, preferred_element_type=jnp.float32)
            t = t * dinv + b[...]
            h = jnp.maximum(t, 0.0) if act else t
        emb = jnp.dot(h, wo[...], preferred_element_type=jnp.float32) + bo[...]
        emb_ref[...] = emb
        wp = wp1[...]
        pa_ref[...] = jnp.dot(emb, wp[:DH], preferred_element_type=jnp.float32) + bp1_ref[...]
        pb = jnp.dot(emb, wp[DH:], preferred_element_type=jnp.float32)
        pbt_ref[...] = pb.T

    return pl.pallas_call(
        body,
        out_shape=(
            jax.ShapeDtypeStruct((N, DH), jnp.float32),
            jax.ShapeDtypeStruct((N, DH), jnp.float32),
            jax.ShapeDtypeStruct((DH, N), jnp.float32),
        ),
    )(A, x, Wc0, bc0, Wc1, bc1, Wc2, bc2, Wout, bout, Wp1, bp1)


def _pairwise_dist(PA, PBT, Wp2, bp2):
    def body(pa_ref, pbt_ref, w_ref, b_ref, out_ref, acc_ref):
        bi = pl.program_id(0)
        p = pl.program_id(1)

        @pl.when(2 * p + 1 >= bi)
        def _():
            bt0 = pbt_ref[:, pl.ds(p * 2 * BLK, BLK)]
            bt1 = pbt_ref[:, pl.ds(p * 2 * BLK + BLK, BLK)]
            wv = w_ref[...]

            def oct_body(r, _):
                a8 = pa_ref[pl.ds(bi * BLK + r * 8, 8), :]
                s0 = [jnp.zeros((8, BLK), jnp.float32) for _ in range(4)]
                s1 = [jnp.zeros((8, BLK), jnp.float32) for _ in range(4)]
                for kk in range(DH // 4):
                    for part in range(4):
                        k = part * (DH // 4) + kk
                        ab = a8[:, k:k + 1]
                        wb = wv[k:k + 1, :]
                        m0 = jnp.maximum(ab + bt0[k:k + 1, :], 0.0)
                        m1 = jnp.maximum(ab + bt1[k:k + 1, :], 0.0)
                        s0[part] = s0[part] + m0 * wb
                        s1[part] = s1[part] + m1 * wb
                acc_ref[pl.ds(r * 8, 8), pl.ds(0, BLK)] = (
                    (s0[0] + s0[1]) + (s0[2] + s0[3]))
                acc_ref[pl.ds(r * 8, 8), pl.ds(BLK, BLK)] = (
                    (s1[0] + s1[1]) + (s1[2] + s1[3]))
                return 0

            lax.fori_loop(0, BLK // 8, oct_body, 0, unroll=2)
            rr = lax.broadcasted_iota(jnp.int32, (BLK, BLK), 0)
            cc = lax.broadcasted_iota(jnp.int32, (BLK, BLK), 1)
            for h in range(2):
                bj = 2 * p + h

                @pl.when(bj >= bi)
                def _(h=h, bj=bj):
                    blk = jax.nn.sigmoid(
                        acc_ref[:, pl.ds(h * BLK, BLK)] + b_ref[0, 0])
                    blk = jnp.where((bi != bj) | (cc > rr), blk, 0.0)

                    @pl.when(bi == bj)
                    def _():
                        out_ref[pl.ds(bi * BLK, BLK), pl.ds(bi * BLK, BLK)] = (
                            blk + blk.T)

                    @pl.when(bi != bj)
                    def _():
                        out_ref[pl.ds(bi * BLK, BLK), pl.ds(bj * BLK, BLK)] = blk
                        out_ref[pl.ds(bj * BLK, BLK), pl.ds(bi * BLK, BLK)] = blk.T

    return pl.pallas_call(
        body,
        grid=(NB, NB // 2),
        in_specs=[
            pl.BlockSpec((N, DH), lambda i, j: (0, 0)),
            pl.BlockSpec((DH, N), lambda i, j: (0, 0)),
            pl.BlockSpec((DH, BLK), lambda i, j: (0, 0)),
            pl.BlockSpec(memory_space=pltpu.SMEM),
        ],
        out_specs=pl.BlockSpec((N, N), lambda i, j: (0, 0)),
        out_shape=jax.ShapeDtypeStruct((N, N), jnp.float32),
        scratch_shapes=[pltpu.VMEM((BLK, 2 * BLK), jnp.float32)],
    )(PA, PBT, jnp.broadcast_to(Wp2[:, 0:1], (DH, BLK)), bp2)


def kernel(x, edge_index, Wc0, bc0, Wc1, bc1, Wc2, bc2, Wout, bout, Wp1, bp1, Wp2, bp2):
    A = _build_adjacency(edge_index)
    emb, PA, PBT = _dense_chain(
        A, x, Wc0, bc0.reshape(1, DH), Wc1, bc1.reshape(1, DH),
        Wc2, bc2.reshape(1, DH), Wout, bout.reshape(1, DH),
        Wp1, bp1.reshape(1, DH))
    dist = _pairwise_dist(PA, PBT, Wp2, bp2.reshape(1, 1))
    return emb, dist


# speedup vs baseline: 1.3251x; 1.0746x over previous
import functools

import jax
import jax.numpy as jnp
from jax import lax
from jax.experimental import pallas as pl
from jax.experimental.pallas import tpu as pltpu
from jax.experimental.pallas import tpu_sc as plsc

N = 1024
E = 32768
DH = 128
NW = 32
RPW = N // NW
LANES = 16
BLK = 128
NB = N // BLK

def _build_adjacency(edge_index):
    mesh = plsc.VectorSubcoreMesh(core_axis_name="c", subcore_axis_name="s")

    @functools.partial(
        pl.kernel,
        out_type=jax.ShapeDtypeStruct((NW, RPW * N), jnp.float32),
        mesh=mesh,
        compiler_params=pltpu.CompilerParams(needs_layout_passes=False),
        scratch_types=[
            pltpu.VMEM((E,), jnp.int32),
            pltpu.VMEM((E,), jnp.int32),
            pltpu.VMEM((RPW * N,), jnp.float32),
        ],
    )
    def adj(edges_hbm, out_hbm, row_v, col_v, slab):
        wid = lax.axis_index("s") * 2 + lax.axis_index("c")
        lo = wid * RPW
        pltpu.sync_copy(edges_hbm.at[0], row_v)
        pltpu.sync_copy(edges_hbm.at[1], col_v)
        zeros16 = jnp.zeros((LANES,), jnp.float32)
        ones16 = jnp.ones((LANES,), jnp.float32)
        def zbody(k, _):
            for u in range(8):
                slab[pl.ds((k * 8 + u) * LANES, LANES)] = zeros16
            return 0
        lax.fori_loop(0, RPW * N // LANES // 8, zbody, 0)
        j16 = lax.iota(jnp.int32, LANES)
        for h in range(RPW // LANES):
            jj = j16 + h * LANES
            plsc.store_scatter(slab, [jj * (N + 1) + lo], ones16)
        def ebody(i, _):
            for u in range(4):
                base = (i * 4 + u) * LANES
                r16 = row_v[pl.ds(base, LANES)]
                c16 = col_v[pl.ds(base, LANES)]
                msk = (c16 >= lo) & (c16 < lo + RPW)
                plsc.addupdate_scatter(slab, [(c16 - lo) * N + r16], ones16,
                                       mask=msk)
            return 0
        lax.fori_loop(0, E // LANES // 4, ebody, 0)
        pltpu.sync_copy(slab, out_hbm.at[wid])

    return adj(edge_index).reshape(N, N)


def _dense_chain(A, x, Wc0, bc0, Wc1, bc1, Wc2, bc2, Wout, bout, Wp1, bp1):
    def body(a_ref, x_ref, w0, b0, w1, b1, w2, b2, wo, bo, wp1, bp1_ref,
             emb_ref, pa_ref, pbt_ref):
        A = a_ref[...]
        deg = jnp.sum(A, axis=1, keepdims=True)
        dinv = lax.rsqrt(deg)
        h = x_ref[...]
        for (w, b, act) in ((w0, b0, True), (w1, b1, True), (w2, b2, False)):
            t = jnp.dot(h, w[...], preferred_element_type=jnp.float32)
            t = t * dinv
            t = jnp.dot(A, t, preferred_element_type=jnp.float32)
            t = t * dinv + b[...]
            h = jnp.maximum(t, 0.0) if act else t
        emb = jnp.dot(h, wo[...], preferred_element_type=jnp.float32) + bo[...]
        emb_ref[...] = emb
        wp = wp1[...]
        pa_ref[...] = jnp.dot(emb, wp[:DH], preferred_element_type=jnp.float32) + bp1_ref[...]
        pb = jnp.dot(emb, wp[DH:], preferred_element_type=jnp.float32)
        pbt_ref[...] = pb.T

    return pl.pallas_call(
        body,
        out_shape=(
            jax.ShapeDtypeStruct((N, DH), jnp.float32),
            jax.ShapeDtypeStruct((N, DH), jnp.float32),
            jax.ShapeDtypeStruct((DH, N), jnp.float32),
        ),
    )(A, x, Wc0, bc0, Wc1, bc1, Wc2, bc2, Wout, bout, Wp1, bp1)


def _pairwise_dist(PA, PBT, Wp2, bp2):
    def body(pa_ref, pbt_ref, w_ref, b_ref, out_ref, acc_ref):
        bi = pl.program_id(0)
        p = pl.program_id(1)

        @pl.when(2 * p + 1 >= bi)
        def _():
            bt0 = pbt_ref[:, pl.ds(p * 2 * BLK, BLK)]
            bt1 = pbt_ref[:, pl.ds(p * 2 * BLK + BLK, BLK)]
            wv = w_ref[...]

            def oct_body(r, _):
                a8 = pa_ref[pl.ds(bi * BLK + r * 8, 8), :]
                s0 = [jnp.zeros((8, BLK), jnp.float32) for _ in range(4)]
                s1 = [jnp.zeros((8, BLK), jnp.float32) for _ in range(4)]
                for kk in range(DH // 4):
                    for part in range(4):
                        k = part * (DH // 4) + kk
                        ab = a8[:, k:k + 1]
                        wb = wv[k:k + 1, :]
                        m0 = jnp.maximum(ab + bt0[k:k + 1, :], 0.0)
                        m1 = jnp.maximum(ab + bt1[k:k + 1, :], 0.0)
                        s0[part] = s0[part] + m0 * wb
                        s1[part] = s1[part] + m1 * wb
                acc_ref[pl.ds(r * 8, 8), pl.ds(0, BLK)] = (
                    (s0[0] + s0[1]) + (s0[2] + s0[3]))
                acc_ref[pl.ds(r * 8, 8), pl.ds(BLK, BLK)] = (
                    (s1[0] + s1[1]) + (s1[2] + s1[3]))
                return 0

            lax.fori_loop(0, BLK // 8, oct_body, 0, unroll=2)
            rr = lax.broadcasted_iota(jnp.int32, (BLK, BLK), 0)
            cc = lax.broadcasted_iota(jnp.int32, (BLK, BLK), 1)
            for h in range(2):
                bj = 2 * p + h

                @pl.when(bj >= bi)
                def _(h=h, bj=bj):
                    blk = jax.nn.sigmoid(
                        acc_ref[:, pl.ds(h * BLK, BLK)] + b_ref[0, 0])
                    blk = jnp.where((bi != bj) | (cc > rr), blk, 0.0)

                    @pl.when(bi == bj)
                    def _():
                        out_ref[pl.ds(bi * BLK, BLK), pl.ds(bi * BLK, BLK)] = (
                            blk + blk.T)

                    @pl.when(bi != bj)
                    def _():
                        out_ref[pl.ds(bi * BLK, BLK), pl.ds(bj * BLK, BLK)] = blk
                        out_ref[pl.ds(bj * BLK, BLK), pl.ds(bi * BLK, BLK)] = blk.T

    return pl.pallas_call(
        body,
        grid=(NB, NB // 2),
        in_specs=[
            pl.BlockSpec((N, DH), lambda i, j: (0, 0)),
            pl.BlockSpec((DH, N), lambda i, j: (0, 0)),
            pl.BlockSpec((DH, BLK), lambda i, j: (0, 0)),
            pl.BlockSpec(memory_space=pltpu.SMEM),
        ],
        out_specs=pl.BlockSpec((N, N), lambda i, j: (0, 0)),
        out_shape=jax.ShapeDtypeStruct((N, N), jnp.float32),
        scratch_shapes=[pltpu.VMEM((BLK, 2 * BLK), jnp.float32)],
    )(PA, PBT, jnp.broadcast_to(Wp2[:, 0:1], (DH, BLK)), bp2)


def kernel(x, edge_index, Wc0, bc0, Wc1, bc1, Wc2, bc2, Wout, bout, Wp1, bp1, Wp2, bp2):
    A = _build_adjacency(edge_index)
    emb, PA, PBT = _dense_chain(
        A, x, Wc0, bc0.reshape(1, DH), Wc1, bc1.reshape(1, DH),
        Wc2, bc2.reshape(1, DH), Wout, bout.reshape(1, DH),
        Wp1, bp1.reshape(1, DH))
    dist = _pairwise_dist(PA, PBT, Wp2, bp2.reshape(1, 1))
    return emb, dist
